# baseline (speedup 1.0000x reference)
import functools

import jax
import jax.numpy as jnp
from jax import lax
from jax.experimental import pallas as pl
from jax.experimental.pallas import tpu as pltpu
from jax.experimental.pallas import tpu_sc as plsc

N_FEATURE = 256
DEPTH = 8
N_INTERNAL = 2**DEPTH - 1
N_LEAF = 2**DEPTH
N_TREE_PAD = 112
N_BATCH = 4096
LANES = 16
NUM_WORKERS = 32
ROWS_PER_W = N_BATCH // NUM_WORKERS


def _tree_kernel_body(x_hbm, feat_hbm, thr_hbm, val_hbm, out_hbm,
                      x_v, feat_v, thr_v, val_v, out_v):
    c = lax.axis_index("c")
    s = lax.axis_index("s")
    wid = s * 2 + c
    base = wid * ROWS_PER_W

    pltpu.sync_copy(x_hbm.at[pl.ds(base, ROWS_PER_W)], x_v)
    pltpu.sync_copy(feat_hbm, feat_v)
    pltpu.sync_copy(thr_hbm, thr_v)
    pltpu.sync_copy(val_hbm, val_v)

    lane = lax.iota(jnp.int32, LANES)
    n_groups = ROWS_PER_W // LANES
    b_vecs = [lane + bg * LANES for bg in range(n_groups)]

    def tree_step(t, accs):
        t_vec = jnp.full((LANES,), t, jnp.int32)
        f0 = plsc.load_gather(feat_v, [t_vec, jnp.zeros((LANES,), jnp.int32)])
        th0 = plsc.load_gather(thr_v, [t_vec, jnp.zeros((LANES,), jnp.int32)])
        nodes = []
        for bg in range(n_groups):
            xv = plsc.load_gather(x_v, [b_vecs[bg], f0])
            nodes.append(1 + (xv > th0).astype(jnp.int32))
        for _ in range(1, DEPTH):
            fs = [plsc.load_gather(feat_v, [t_vec, n]) for n in nodes]
            ths = [plsc.load_gather(thr_v, [t_vec, n]) for n in nodes]
            xvs = [plsc.load_gather(x_v, [b_vecs[bg], fs[bg]])
                   for bg in range(n_groups)]
            nodes = [2 * nodes[bg] + 1 + (xvs[bg] > ths[bg]).astype(jnp.int32)
                     for bg in range(n_groups)]
        return tuple(
            accs[bg] + plsc.load_gather(val_v, [t_vec, nodes[bg] - N_INTERNAL])
            for bg in range(n_groups))

    accs0 = tuple(jnp.zeros((LANES,), jnp.float32) for _ in range(n_groups))
    accs = lax.fori_loop(0, N_TREE_PAD, tree_step, accs0)
    for bg in range(n_groups):
        out_v[pl.ds(bg * LANES, LANES)] = 1.0 / (1.0 + jnp.exp(-accs[bg]))

    pltpu.sync_copy(out_v, out_hbm.at[pl.ds(base, ROWS_PER_W)])


@functools.partial(jax.jit, static_argnames=())
def _run_sc(x, feat_p, thr_p, val_p):
    mesh = plsc.VectorSubcoreMesh(core_axis_name="c", subcore_axis_name="s")
    call = pl.kernel(
        _tree_kernel_body,
        out_type=jax.ShapeDtypeStruct((N_BATCH,), jnp.float32),
        mesh=mesh,
        scratch_types=[
            pltpu.VMEM((ROWS_PER_W, N_FEATURE), jnp.float32),
            pltpu.VMEM((N_TREE_PAD, N_LEAF), jnp.int32),
            pltpu.VMEM((N_TREE_PAD, N_LEAF), jnp.float32),
            pltpu.VMEM((N_TREE_PAD, N_LEAF), jnp.float32),
            pltpu.VMEM((ROWS_PER_W,), jnp.float32),
        ],
        compiler_params=pltpu.CompilerParams(use_tc_tiling_on_sc=False,
                                             needs_layout_passes=False),
    )
    return call(x, feat_p, thr_p, val_p)


def kernel(x, feature, threshold, children_left, children_right, value):
    del children_left, children_right
    n_batch, _ = x.shape
    n_tree, _ = feature.shape
    feat_i = jnp.maximum(feature[:, :N_INTERNAL], 0)
    feat_p = jnp.pad(feat_i, ((0, N_TREE_PAD - n_tree), (0, N_LEAF - N_INTERNAL)))
    thr_p = jnp.pad(threshold[:, :N_INTERNAL],
                    ((0, N_TREE_PAD - n_tree), (0, N_LEAF - N_INTERNAL)))
    val_p = jnp.pad(value[:, N_INTERNAL:, 0], ((0, N_TREE_PAD - n_tree), (0, 0)))
    out = _run_sc(x, feat_p, thr_p, val_p.astype(jnp.float32))
    return out.reshape(n_batch, 1)


# speedup vs baseline: 1.0369x; 1.0369x over previous
import functools

import jax
import jax.numpy as jnp
from jax import lax
from jax.experimental import pallas as pl
from jax.experimental.pallas import tpu as pltpu
from jax.experimental.pallas import tpu_sc as plsc

N_FEATURE = 256
DEPTH = 8
N_INTERNAL = 2**DEPTH - 1
N_LEAF = 2**DEPTH
N_TREE_PAD = 112
N_BATCH = 4096
LANES = 16
NUM_WORKERS = 32
ROWS_PER_W = N_BATCH // NUM_WORKERS


def _tree_kernel_body(x_hbm, feat_hbm, thr_hbm, val_hbm, out_hbm,
                      x_v, feat_v, thr_v, val_v, out_v):
    c = lax.axis_index("c")
    s = lax.axis_index("s")
    wid = s * 2 + c
    base = wid * ROWS_PER_W

    pltpu.sync_copy(x_hbm.at[pl.ds(base, ROWS_PER_W)], x_v)
    pltpu.sync_copy(feat_hbm, feat_v)
    pltpu.sync_copy(thr_hbm, thr_v)
    pltpu.sync_copy(val_hbm, val_v)

    lane = lax.iota(jnp.int32, LANES)
    n_groups = ROWS_PER_W // LANES
    b_vecs = [lane + bg * LANES for bg in range(n_groups)]

    def tree_step(t, accs):
        t_vec = jnp.full((LANES,), t, jnp.int32)
        ft16 = feat_v[t, pl.ds(0, LANES)]
        th16 = thr_v[t, pl.ds(0, LANES)]
        nodes = [jnp.zeros((LANES,), jnp.int32)] * n_groups
        for d in range(DEPTH):
            if d < 4:
                fs = [jnp.take_along_axis(ft16, n, axis=0) for n in nodes]
                ths = [jnp.take_along_axis(th16, n, axis=0) for n in nodes]
            else:
                fs = [plsc.load_gather(feat_v, [t_vec, n]) for n in nodes]
                ths = [plsc.load_gather(thr_v, [t_vec, n]) for n in nodes]
            xvs = [plsc.load_gather(x_v, [b_vecs[bg], fs[bg]])
                   for bg in range(n_groups)]
            nodes = [2 * nodes[bg] + 1 + (xvs[bg] > ths[bg]).astype(jnp.int32)
                     for bg in range(n_groups)]
        return tuple(
            accs[bg] + plsc.load_gather(val_v, [t_vec, nodes[bg] - N_INTERNAL])
            for bg in range(n_groups))

    accs0 = tuple(jnp.zeros((LANES,), jnp.float32) for _ in range(n_groups))
    accs = lax.fori_loop(0, N_TREE_PAD, tree_step, accs0)
    for bg in range(n_groups):
        out_v[pl.ds(bg * LANES, LANES)] = 1.0 / (1.0 + jnp.exp(-accs[bg]))

    pltpu.sync_copy(out_v, out_hbm.at[pl.ds(base, ROWS_PER_W)])


@functools.partial(jax.jit, static_argnames=())
def _run_sc(x, feat_p, thr_p, val_p):
    mesh = plsc.VectorSubcoreMesh(core_axis_name="c", subcore_axis_name="s")
    call = pl.kernel(
        _tree_kernel_body,
        out_type=jax.ShapeDtypeStruct((N_BATCH,), jnp.float32),
        mesh=mesh,
        scratch_types=[
            pltpu.VMEM((ROWS_PER_W, N_FEATURE), jnp.float32),
            pltpu.VMEM((N_TREE_PAD, N_LEAF), jnp.int32),
            pltpu.VMEM((N_TREE_PAD, N_LEAF), jnp.float32),
            pltpu.VMEM((N_TREE_PAD, N_LEAF), jnp.float32),
            pltpu.VMEM((ROWS_PER_W,), jnp.float32),
        ],
        compiler_params=pltpu.CompilerParams(use_tc_tiling_on_sc=False,
                                             needs_layout_passes=False),
    )
    return call(x, feat_p, thr_p, val_p)


def kernel(x, feature, threshold, children_left, children_right, value):
    del children_left, children_right
    n_batch, _ = x.shape
    n_tree, _ = feature.shape
    feat_i = jnp.maximum(feature[:, :N_INTERNAL], 0)
    feat_p = jnp.pad(feat_i, ((0, N_TREE_PAD - n_tree), (0, N_LEAF - N_INTERNAL)))
    thr_p = jnp.pad(threshold[:, :N_INTERNAL],
                    ((0, N_TREE_PAD - n_tree), (0, N_LEAF - N_INTERNAL)))
    val_p = jnp.pad(value[:, N_INTERNAL:, 0], ((0, N_TREE_PAD - n_tree), (0, 0)))
    out = _run_sc(x, feat_p, thr_p, val_p.astype(jnp.float32))
    return out.reshape(n_batch, 1)


# speedup vs baseline: 1.5943x; 1.5376x over previous
import functools

import jax
import jax.numpy as jnp
from jax import lax
from jax.experimental import pallas as pl
from jax.experimental.pallas import tpu as pltpu
from jax.experimental.pallas import tpu_sc as plsc

N_FEATURE = 256
DEPTH = 8
N_INTERNAL = 2**DEPTH - 1
N_LEAF = 2**DEPTH
N_TREE_PAD = 112
N_BATCH = 4096
LANES = 16
NUM_WORKERS = 32
ROWS_PER_W = N_BATCH // NUM_WORKERS


def _tree_kernel_body(xt_hbm, feat_hbm, thr_hbm, val_hbm, out_hbm,
                      x_v, feat_v, thr_v, val_v, out_v, sem):
    c = lax.axis_index("c")
    s = lax.axis_index("s")
    wid = s * 2 + c
    base = wid * ROWS_PER_W

    cp = [pltpu.async_copy(xt_hbm.at[:, pl.ds(base, ROWS_PER_W)], x_v, sem),
          pltpu.async_copy(feat_hbm, feat_v, sem),
          pltpu.async_copy(thr_hbm, thr_v, sem),
          pltpu.async_copy(val_hbm, val_v, sem)]
    for c_ in cp:
        c_.wait()

    lane = lax.iota(jnp.int32, LANES)
    n_groups = ROWS_PER_W // LANES
    b_vecs = [lane + bg * LANES for bg in range(n_groups)]

    def tree_step(t, accs):
        t_vec = jnp.full((LANES,), t, jnp.int32)
        ft16 = feat_v[t, pl.ds(0, LANES)]
        th16 = thr_v[t, pl.ds(0, LANES)]
        nodes = [jnp.zeros((LANES,), jnp.int32)] * n_groups
        for d in range(DEPTH):
            if d < 4:
                fs = [jnp.take_along_axis(ft16, n, axis=0) for n in nodes]
                ths = [jnp.take_along_axis(th16, n, axis=0) for n in nodes]
            else:
                fs = [plsc.load_gather(feat_v, [t_vec, n]) for n in nodes]
                ths = [plsc.load_gather(thr_v, [t_vec, n]) for n in nodes]
            xvs = [plsc.load_gather(x_v, [fs[bg], b_vecs[bg]])
                   for bg in range(n_groups)]
            nodes = [2 * nodes[bg] + 1 + (xvs[bg] > ths[bg]).astype(jnp.int32)
                     for bg in range(n_groups)]
        return tuple(
            accs[bg] + plsc.load_gather(val_v, [t_vec, nodes[bg] - N_INTERNAL])
            for bg in range(n_groups))

    accs0 = tuple(jnp.zeros((LANES,), jnp.float32) for _ in range(n_groups))
    accs = lax.fori_loop(0, N_TREE_PAD, tree_step, accs0)
    for bg in range(n_groups):
        out_v[pl.ds(bg * LANES, LANES)] = 1.0 / (1.0 + jnp.exp(-accs[bg]))

    pltpu.sync_copy(out_v, out_hbm.at[pl.ds(base, ROWS_PER_W)])


@functools.partial(jax.jit, static_argnames=())
def _run_sc(xt, feat_p, thr_p, val_p):
    mesh = plsc.VectorSubcoreMesh(core_axis_name="c", subcore_axis_name="s")
    call = pl.kernel(
        _tree_kernel_body,
        out_type=jax.ShapeDtypeStruct((N_BATCH,), jnp.float32),
        mesh=mesh,
        scratch_types=[
            pltpu.VMEM((N_FEATURE, ROWS_PER_W), jnp.float32),
            pltpu.VMEM((N_TREE_PAD, N_LEAF), jnp.int32),
            pltpu.VMEM((N_TREE_PAD, N_LEAF), jnp.float32),
            pltpu.VMEM((N_TREE_PAD, N_LEAF), jnp.float32),
            pltpu.VMEM((ROWS_PER_W,), jnp.float32),
            pltpu.SemaphoreType.DMA,
        ],
        compiler_params=pltpu.CompilerParams(use_tc_tiling_on_sc=False,
                                             needs_layout_passes=False),
    )
    return call(xt, feat_p, thr_p, val_p)


def kernel(x, feature, threshold, children_left, children_right, value):
    del children_left, children_right
    n_batch, _ = x.shape
    n_tree, _ = feature.shape
    feat_i = jnp.maximum(feature[:, :N_INTERNAL], 0)
    feat_p = jnp.pad(feat_i, ((0, N_TREE_PAD - n_tree), (0, N_LEAF - N_INTERNAL)))
    thr_p = jnp.pad(threshold[:, :N_INTERNAL],
                    ((0, N_TREE_PAD - n_tree), (0, N_LEAF - N_INTERNAL)))
    val_p = jnp.pad(value[:, N_INTERNAL:, 0], ((0, N_TREE_PAD - n_tree), (0, 0)))
    out = _run_sc(x.T, feat_p, thr_p, val_p.astype(jnp.float32))
    return out.reshape(n_batch, 1)


# speedup vs baseline: 1.6040x; 1.0061x over previous
import functools

import jax
import jax.numpy as jnp
from jax import lax
from jax.experimental import pallas as pl
from jax.experimental.pallas import tpu as pltpu
from jax.experimental.pallas import tpu_sc as plsc

N_FEATURE = 256
DEPTH = 8
N_INTERNAL = 2**DEPTH - 1
N_LEAF = 2**DEPTH
N_TREE = 100
N_BATCH = 4096
LANES = 16
NUM_WORKERS = 32
ROWS_PER_W = N_BATCH // NUM_WORKERS
REG_DEPTH = 6


def _tree_kernel_body(xt_hbm, feat_hbm, thr_hbm, val_hbm, out_hbm,
                      x_v, feat_v, thr_v, val_v, out_v, sem):
    c = lax.axis_index("c")
    s = lax.axis_index("s")
    wid = s * 2 + c
    base = wid * ROWS_PER_W

    cp = [pltpu.async_copy(xt_hbm.at[:, pl.ds(base, ROWS_PER_W)], x_v, sem),
          pltpu.async_copy(feat_hbm, feat_v, sem),
          pltpu.async_copy(thr_hbm, thr_v, sem),
          pltpu.async_copy(val_hbm, val_v, sem)]
    for c_ in cp:
        c_.wait()

    lane = lax.iota(jnp.int32, LANES)
    n_groups = ROWS_PER_W // LANES
    b_vecs = [lane + bg * LANES for bg in range(n_groups)]
    n_reg_nodes = 2**REG_DEPTH - 1
    n_reg_vecs = (n_reg_nodes + LANES) // LANES

    def reg_lookup(vregs, node, klo, khi):
        out = jnp.take_along_axis(vregs[klo],
                                  jnp.clip(node - klo * LANES, 0, LANES - 1),
                                  axis=0)
        for k in range(klo + 1, khi + 1):
            idx = jnp.clip(node - k * LANES, 0, LANES - 1)
            out = jnp.where(node >= k * LANES,
                            jnp.take_along_axis(vregs[k], idx, axis=0), out)
        return out

    @plsc.parallel_loop(0, N_TREE, carry=tuple(
        jnp.zeros((LANES,), jnp.float32) for _ in range(n_groups)))
    def accs(t, accs):
        t_vec = jnp.full((LANES,), t, jnp.int32)
        ftr = [feat_v[t, pl.ds(k * LANES, LANES)] for k in range(n_reg_vecs)]
        thr = [thr_v[t, pl.ds(k * LANES, LANES)] for k in range(n_reg_vecs)]
        nodes = [jnp.zeros((LANES,), jnp.int32)] * n_groups
        for d in range(DEPTH):
            if d < 4:
                fs = [jnp.take_along_axis(ftr[0], n, axis=0) for n in nodes]
                ths = [jnp.take_along_axis(thr[0], n, axis=0) for n in nodes]
            elif d < REG_DEPTH:
                klo, khi = (2**d - 1) // LANES, (2**(d + 1) - 2) // LANES
                fs = [reg_lookup(ftr, n, klo, khi) for n in nodes]
                ths = [reg_lookup(thr, n, klo, khi) for n in nodes]
            else:
                fs = [plsc.load_gather(feat_v, [t_vec, n]) for n in nodes]
                ths = [plsc.load_gather(thr_v, [t_vec, n]) for n in nodes]
            xvs = [plsc.load_gather(x_v, [fs[bg], b_vecs[bg]])
                   for bg in range(n_groups)]
            nodes = [2 * nodes[bg] + 1 + (xvs[bg] > ths[bg]).astype(jnp.int32)
                     for bg in range(n_groups)]
        return tuple(
            accs[bg] + plsc.load_gather(val_v, [t_vec, nodes[bg] - N_INTERNAL])
            for bg in range(n_groups))
    for bg in range(n_groups):
        out_v[pl.ds(bg * LANES, LANES)] = 1.0 / (1.0 + jnp.exp(-accs[bg]))

    pltpu.sync_copy(out_v, out_hbm.at[pl.ds(base, ROWS_PER_W)])


@functools.partial(jax.jit, static_argnames=())
def _run_sc(xt, feat_p, thr_p, val_p):
    mesh = plsc.VectorSubcoreMesh(core_axis_name="c", subcore_axis_name="s")
    call = pl.kernel(
        _tree_kernel_body,
        out_type=jax.ShapeDtypeStruct((N_BATCH,), jnp.float32),
        mesh=mesh,
        scratch_types=[
            pltpu.VMEM((N_FEATURE, ROWS_PER_W), jnp.float32),
            pltpu.VMEM((N_TREE, N_LEAF), jnp.int32),
            pltpu.VMEM((N_TREE, N_LEAF), jnp.float32),
            pltpu.VMEM((N_TREE, N_LEAF), jnp.float32),
            pltpu.VMEM((ROWS_PER_W,), jnp.float32),
            pltpu.SemaphoreType.DMA,
        ],
        compiler_params=pltpu.CompilerParams(use_tc_tiling_on_sc=False,
                                             needs_layout_passes=False),
    )
    return call(xt, feat_p, thr_p, val_p)


def kernel(x, feature, threshold, children_left, children_right, value):
    del children_left, children_right
    n_batch, _ = x.shape
    n_tree, _ = feature.shape
    del n_tree
    feat_p = jnp.pad(jnp.maximum(feature[:, :N_INTERNAL], 0),
                     ((0, 0), (0, N_LEAF - N_INTERNAL)))
    thr_p = jnp.pad(threshold[:, :N_INTERNAL],
                    ((0, 0), (0, N_LEAF - N_INTERNAL)))
    val_p = value[:, N_INTERNAL:, 0]
    out = _run_sc(x.T, feat_p, thr_p, val_p.astype(jnp.float32))
    return out.reshape(n_batch, 1)
